# baseline (speedup 1.0000x reference)
import functools

import jax
import jax.numpy as jnp
from jax import lax
from jax.experimental import pallas as pl
from jax.experimental.pallas import tpu as pltpu
from jax.experimental.pallas import tpu_sc as plsc

N_MESH = 10000
E = 320000
D = 128
EPS = 1e-5

_NC = 2
_NS = 16
_NW = _NC * _NS
_EW = E // _NW
_CH = 128
_NPAD = 10240
_BR = _NPAD // _NS
_DR = 64


def _ln(z, g, b):
    m = jnp.mean(z, axis=-1, keepdims=True)
    v = jnp.mean((z - m) ** 2, axis=-1, keepdims=True)
    return (z - m) / jnp.sqrt(v + EPS) * g + b


def _silu(x):
    return x * jax.nn.sigmoid(x)


def _rows(bs, nd=D):
    return pl.BlockSpec((bs, nd), lambda i: (i, 0))


def _bcast(shape):
    return pl.BlockSpec(shape, lambda i: tuple(0 for _ in shape))


def _pack_cols(z):
    Dh = D // 2
    lo_f = z[:, :Dh].astype(jnp.bfloat16).astype(jnp.float32)
    hi_f = z[:, Dh:].astype(jnp.bfloat16).astype(jnp.float32)
    lo_b = jax.lax.shift_right_logical(
        jax.lax.bitcast_convert_type(lo_f, jnp.int32), 16)
    hi_b = jax.lax.bitcast_convert_type(hi_f, jnp.int32) & jnp.int32(-65536)
    return hi_b | lo_b


def _proj_body(g_ref, m_ref, B_ref, C_ref, pg_ref, pm_ref):
    pg_ref[...] = _pack_cols(jnp.dot(g_ref[...], B_ref[...],
                                     preferred_element_type=jnp.float32))
    pm_ref[...] = _pack_cols(jnp.dot(m_ref[...], C_ref[...],
                                     preferred_element_type=jnp.float32))


def _project(grid10k, mesh, B, C, bs=2000):
    n = N_MESH // bs
    return pl.pallas_call(
        _proj_body,
        grid=(n,),
        in_specs=[_rows(bs), _rows(bs), _bcast((D, D)), _bcast((D, D))],
        out_specs=[_rows(bs, D // 2), _rows(bs, D // 2)],
        out_shape=[jax.ShapeDtypeStruct((N_MESH, D // 2), jnp.int32)] * 2,
        compiler_params=pltpu.CompilerParams(
            dimension_semantics=("arbitrary",)),
    )(grid10k, mesh, B, C)


def _lo_bf16(q):
    return jax.lax.bitcast_convert_type(q << 16, jnp.float32)


def _hi_bf16(q):
    return jax.lax.bitcast_convert_type(q & jnp.int32(-65536), jnp.float32)


def _edge_body(e_ref, gi_ref, A_ref, b1_ref, W2_ref, b2_ref,
               g_ref, bt_ref, y_ref):
    Dh = D // 2
    p = gi_ref[...]
    pg_i = p[:, :Dh]
    pm_i = p[:, Dh:]
    e = e_ref[...]
    h_lo = (jnp.dot(e, A_ref[:, :Dh], preferred_element_type=jnp.float32)
            + _lo_bf16(pg_i) + _lo_bf16(pm_i) + b1_ref[:, :Dh])
    h_hi = (jnp.dot(e, A_ref[:, Dh:], preferred_element_type=jnp.float32)
            + _hi_bf16(pg_i) + _hi_bf16(pm_i) + b1_ref[:, Dh:])
    z = (jnp.dot(_silu(h_lo), W2_ref[:Dh, :],
                 preferred_element_type=jnp.float32)
         + jnp.dot(_silu(h_hi), W2_ref[Dh:, :],
                   preferred_element_type=jnp.float32))
    y_ref[...] = _ln(z + b2_ref[...], g_ref[...], bt_ref[...])


def _edge_mlp(e, gi, A, b1, W2, b2, g, bt, lo, bs=1600):
    n = gi.shape[0] // bs
    blk0 = lo // bs
    return pl.pallas_call(
        _edge_body,
        grid=(n,),
        in_specs=[pl.BlockSpec((bs, D), lambda i: (i + blk0, 0)), _rows(bs),
                  _bcast((D, D)), _bcast((1, D)), _bcast((D, D)),
                  _bcast((1, D)), _bcast((1, D)), _bcast((1, D))],
        out_specs=_rows(bs),
        out_shape=jax.ShapeDtypeStruct((gi.shape[0], D), jnp.float32),
        compiler_params=pltpu.CompilerParams(
            dimension_semantics=("arbitrary",)),
    )(e, gi, A, b1.reshape(1, D), W2, b2.reshape(1, D),
      g.reshape(1, D), bt.reshape(1, D))


def _node_body(x_ref, W1_ref, b1_ref, W2_ref, b2_ref, g_ref, bt_ref, o_ref):
    x = x_ref[...]
    h = jnp.dot(x, W1_ref[...], preferred_element_type=jnp.float32)
    h = _silu(h + b1_ref[...])
    z = jnp.dot(h, W2_ref[...], preferred_element_type=jnp.float32)
    o_ref[...] = x + _ln(z + b2_ref[...], g_ref[...], bt_ref[...])


def _grid_mlp(x, W1, b1, W2, b2, g, bt, bs=2000):
    n = x.shape[0] // bs
    return pl.pallas_call(
        _node_body,
        grid=(n,),
        in_specs=[_rows(bs), _bcast((D, D)), _bcast((1, D)), _bcast((D, D)),
                  _bcast((1, D)), _bcast((1, D)), _bcast((1, D))],
        out_specs=_rows(bs),
        out_shape=jax.ShapeDtypeStruct(x.shape, jnp.float32),
        compiler_params=pltpu.CompilerParams(
            dimension_semantics=("arbitrary",)),
    )(x, W1, b1.reshape(1, D), W2, b2.reshape(1, D), g.reshape(1, D),
      bt.reshape(1, D))


def _make_mesh_body(nagg):
    def body(*refs):
        agg_refs = refs[:nagg]
        (x_ref, W1a_ref, W1b_ref, b1_ref, W2_ref, b2_ref, g_ref, bt_ref,
         o_ref) = refs[nagg:]
        a = agg_refs[0][0] + agg_refs[0][1]
        for r in agg_refs[1:]:
            a = a + r[0] + r[1]
        x = x_ref[...]
        h = (jnp.dot(a, W1a_ref[...], preferred_element_type=jnp.float32)
             + jnp.dot(x, W1b_ref[...], preferred_element_type=jnp.float32))
        h = _silu(h + b1_ref[...])
        z = jnp.dot(h, W2_ref[...], preferred_element_type=jnp.float32)
        o_ref[...] = x + _ln(z + b2_ref[...], g_ref[...], bt_ref[...])
    return body


def _mesh_mlp(aggs, x, W1a, W1b, b1, W2, b2, g, bt, bs=2000):
    n = N_MESH // bs
    agg_spec = pl.BlockSpec((2, bs, D), lambda i: (0, i, 0))
    return pl.pallas_call(
        _make_mesh_body(len(aggs)),
        grid=(n,),
        in_specs=[agg_spec] * len(aggs) +
                 [_rows(bs),
                  _bcast((D, D)), _bcast((D, D)), _bcast((1, D)),
                  _bcast((D, D)), _bcast((1, D)), _bcast((1, D)),
                  _bcast((1, D))],
        out_specs=_rows(bs),
        out_shape=jax.ShapeDtypeStruct((N_MESH, D), jnp.float32),
        compiler_params=pltpu.CompilerParams(
            dimension_semantics=("arbitrary",)),
    )(*aggs, x, W1a, W1b, b1.reshape(1, D), W2, b2.reshape(1, D),
      g.reshape(1, D), bt.reshape(1, D))


_CHG = 256


def _sc_gather(pg, pm, ei, lo, size):
    nck = size // _CHG
    Dh = D // 2
    mesh = plsc.VectorSubcoreMesh(core_axis_name="c", subcore_axis_name="s")

    @functools.partial(
        pl.kernel,
        out_type=jax.ShapeDtypeStruct((size, D), jnp.int32),
        mesh=mesh,
        scratch_types=[
            pltpu.VMEM((2, _CHG), jnp.int32),
            pltpu.VMEM((2, _CHG), jnp.int32),
            pltpu.VMEM((_CHG, Dh), jnp.int32),
            pltpu.VMEM((_CHG, Dh), jnp.int32),
            pltpu.SemaphoreType.DMA,
            pltpu.SemaphoreType.DMA,
        ],
        compiler_params=pltpu.CompilerParams(use_tc_tiling_on_sc=False),
    )
    def k(pg_hbm, pm_hbm, ei_hbm, go_hbm, ix0_v, ix1_v, rg_v, rm_v,
          sem_i, sem_g):
        wid = lax.axis_index("s") * _NC + lax.axis_index("c")
        nw = (nck - wid + _NW - 1) // _NW

        pltpu.async_copy(ei_hbm.at[:, pl.ds(lo + wid * _CHG, _CHG)], ix0_v,
                         sem_i)

        def phase(j, cur, nxt):
            @pl.when(j < nw)
            def _():
                ci = wid + j * _NW
                off = ci * _CHG
                pltpu.make_async_copy(
                    ei_hbm.at[:, pl.ds(lo + off, _CHG)], cur, sem_i).wait()

                @pl.when(j + 1 < nw)
                def _():
                    off2 = (ci + _NW) * _CHG
                    pltpu.async_copy(
                        ei_hbm.at[:, pl.ds(lo + off2, _CHG)], nxt, sem_i)

                cs = [
                    pltpu.async_copy(pg_hbm.at[cur.at[0, pl.ds(0, 128)]],
                                     rg_v.at[pl.ds(0, 128)], sem_g),
                    pltpu.async_copy(pg_hbm.at[cur.at[0, pl.ds(128, 128)]],
                                     rg_v.at[pl.ds(128, 128)], sem_g),
                    pltpu.async_copy(pm_hbm.at[cur.at[1, pl.ds(0, 128)]],
                                     rm_v.at[pl.ds(0, 128)], sem_g),
                    pltpu.async_copy(pm_hbm.at[cur.at[1, pl.ds(128, 128)]],
                                     rm_v.at[pl.ds(128, 128)], sem_g),
                ]
                for c in cs:
                    c.wait()
                pltpu.sync_copy(rg_v, go_hbm.at[pl.ds(off, _CHG),
                                                pl.ds(0, Dh)])
                pltpu.sync_copy(rm_v, go_hbm.at[pl.ds(off, _CHG),
                                                pl.ds(Dh, Dh)])

        def pair(i2, carry):
            phase(i2 * 2, ix0_v, ix1_v)
            phase(i2 * 2 + 1, ix1_v, ix0_v)
            return carry

        lax.fori_loop(0, (nw + 1) // 2, pair, 0)

    return k(pg, pm, ei)


def _sc_scatter(ys_los, dst):
    nys = len(ys_los)
    ncks = [y.shape[0] // _CH for y, _ in ys_los]
    los = [lo for _, lo in ys_los]
    mesh = plsc.VectorSubcoreMesh(core_axis_name="c", subcore_axis_name="s")

    @functools.partial(
        pl.kernel,
        out_type=jax.ShapeDtypeStruct((_NC, _NS, _BR, D), jnp.float32),
        mesh=mesh,
        scratch_types=[
            pltpu.VMEM((_CH,), jnp.int32),
            pltpu.VMEM((_CH,), jnp.int32),
            pltpu.VMEM((_CH, D), jnp.float32),
            pltpu.VMEM((_CH, D), jnp.float32),
            pltpu.VMEM((_DR, D), jnp.float32),
            pltpu.VMEM_SHARED((_NPAD, D), jnp.float32),
            pltpu.SemaphoreType.DMA,
            pltpu.SemaphoreType.DMA,
            pltpu.SemaphoreType.DMA,
            pltpu.SemaphoreType.DMA,
        ],
    )
    def k(*refs):
        y_hbms = refs[:nys]
        (dst_hbm, out_hbm, di0_v, di1_v, rw0_v, rw1_v, buf_v,
         acc_sh, semL0, semL1, semA0, semA1) = refs[nys:]
        c = lax.axis_index("c")
        s = lax.axis_index("s")
        wid = s * _NC + c

        zero = jnp.zeros((16,), jnp.float32)

        def zrow(r, carry):
            for j in range(D // 16):
                buf_v[r, pl.ds(j * 16, 16)] = zero
            return carry

        lax.fori_loop(0, _DR, zrow, 0)
        for j in range(_BR // _DR):
            pltpu.sync_copy(buf_v, acc_sh.at[pl.ds(s * _BR + j * _DR, _DR)])
        plsc.subcore_barrier()

        bufs = ((di0_v, rw0_v, semL0, semA0), (di1_v, rw1_v, semL1, semA1))

        def wait_A(b):
            pltpu.make_async_copy(b[1], acc_sh.at[b[0]], b[3]).wait()

        for y_hbm, nck, lo in zip(y_hbms, ncks, los):
            nw = (nck - wid + _NW - 1) // _NW

            def fire_L(j, b):
                off = (wid + j * _NW) * _CH
                pltpu.async_copy(dst_hbm.at[pl.ds(lo + off, _CH)], b[0], b[2])
                pltpu.async_copy(y_hbm.at[pl.ds(off, _CH)], b[1], b[2])

            def wait_L(b):
                pltpu.make_async_copy(
                    dst_hbm.at[pl.ds(lo, _CH)], b[0], b[2]).wait()
                pltpu.make_async_copy(
                    y_hbm.at[pl.ds(0, _CH)], b[1], b[2]).wait()

            fire_L(0, bufs[0])

            def phase(j, p):
                cur = bufs[p]
                oth = bufs[1 - p]

                @pl.when(j < nw)
                def _():
                    wait_L(cur)
                    pltpu.async_copy(cur[1], acc_sh.at[cur[0]], cur[3],
                                     add=True)

                    @pl.when(j + 1 < nw)
                    def _():
                        @pl.when(j >= 1)
                        def _():
                            wait_A(oth)

                        fire_L(j + 1, oth)

            def pair(i2, carry):
                phase(i2 * 2, 0)
                phase(i2 * 2 + 1, 1)
                return carry

            lax.fori_loop(0, (nw + 1) // 2, pair, 0)

            wait_A(bufs[0])
            wait_A(bufs[1])

        plsc.subcore_barrier()

        for j in range(_BR // _DR):
            pltpu.sync_copy(acc_sh.at[pl.ds(s * _BR + j * _DR, _DR)], buf_v)
            pltpu.sync_copy(buf_v, out_hbm.at[c, s, pl.ds(j * _DR, _DR)])

    return k(*[y for y, _ in ys_los], dst).reshape(_NC, _NPAD, D)


def kernel(g2m_efeat, grid_nfeat, mesh_nfeat, edge_index,
           eW1, eb1, eW2, eb2, eg, ebt,
           sW1, sb1, sW2, sb2, sg, sbt,
           dW1, db1, dW2, db2, dg, dbt):
    src = edge_index[0]
    dst = edge_index[1]

    A = eW1[:D]
    B = eW1[D:2 * D]
    C = eW1[2 * D:]

    pgi, pmi = _project(grid_nfeat[:N_MESH], mesh_nfeat, B, C)
    splits = (76800, 76800, 76800, 89600)
    lo = 0
    ys = []
    for sz in splits:
        gi_k = _sc_gather(pgi, pmi, edge_index, lo, sz)
        y_k = _edge_mlp(g2m_efeat, gi_k, A, eb1, eW2, eb2, eg, ebt, lo)
        ys.append((y_k, lo))
        lo += sz
    aggs = [_sc_scatter(ys[:2], dst), _sc_scatter(ys[2:], dst)]
    mesh_new = _mesh_mlp(aggs, mesh_nfeat, dW1[:D], dW1[D:], db1,
                         dW2, db2, dg, dbt)
    grid_new = _grid_mlp(grid_nfeat, sW1, sb1, sW2, sb2, sg, sbt)
    return (grid_new, mesh_new)


# speedup vs baseline: 1.1394x; 1.1394x over previous
import functools

import jax
import jax.numpy as jnp
from jax import lax
from jax.experimental import pallas as pl
from jax.experimental.pallas import tpu as pltpu
from jax.experimental.pallas import tpu_sc as plsc

N_MESH = 10000
E = 320000
D = 128
EPS = 1e-5

_NC = 2
_NS = 16
_NW = _NC * _NS
_EW = E // _NW
_CH = 128
_NPAD = 10240
_BR = _NPAD // _NS
_DR = 64


def _ln(z, g, b):
    m = jnp.mean(z, axis=-1, keepdims=True)
    v = jnp.mean((z - m) ** 2, axis=-1, keepdims=True)
    return (z - m) / jnp.sqrt(v + EPS) * g + b


def _silu(x):
    return x * jax.nn.sigmoid(x)


def _rows(bs, nd=D):
    return pl.BlockSpec((bs, nd), lambda i: (i, 0))


def _bcast(shape):
    return pl.BlockSpec(shape, lambda i: tuple(0 for _ in shape))


def _pack_cols(z):
    Dh = D // 2
    lo_f = z[:, :Dh].astype(jnp.bfloat16).astype(jnp.float32)
    hi_f = z[:, Dh:].astype(jnp.bfloat16).astype(jnp.float32)
    lo_b = jax.lax.shift_right_logical(
        jax.lax.bitcast_convert_type(lo_f, jnp.int32), 16)
    hi_b = jax.lax.bitcast_convert_type(hi_f, jnp.int32) & jnp.int32(-65536)
    return hi_b | lo_b


def _proj_body(g_ref, m_ref, B_ref, C_ref, pg_ref, pm_ref):
    pg_ref[...] = _pack_cols(jnp.dot(g_ref[...], B_ref[...],
                                     preferred_element_type=jnp.float32))
    pm_ref[...] = _pack_cols(jnp.dot(m_ref[...], C_ref[...],
                                     preferred_element_type=jnp.float32))


def _project(grid10k, mesh, B, C, bs=2000):
    n = N_MESH // bs
    return pl.pallas_call(
        _proj_body,
        grid=(n,),
        in_specs=[_rows(bs), _rows(bs), _bcast((D, D)), _bcast((D, D))],
        out_specs=[_rows(bs, D // 2), _rows(bs, D // 2)],
        out_shape=[jax.ShapeDtypeStruct((N_MESH, D // 2), jnp.int32)] * 2,
        compiler_params=pltpu.CompilerParams(
            dimension_semantics=("arbitrary",)),
    )(grid10k, mesh, B, C)


def _lo_bf16(q):
    return jax.lax.bitcast_convert_type(q << 16, jnp.float32)


def _hi_bf16(q):
    return jax.lax.bitcast_convert_type(q & jnp.int32(-65536), jnp.float32)


def _edge_body(e_ref, gi_ref, A_ref, b1_ref, W2_ref, b2_ref,
               g_ref, bt_ref, y_ref):
    Dh = D // 2
    p = gi_ref[...]
    pg_i = p[:, :Dh]
    pm_i = p[:, Dh:]
    e = e_ref[...]
    h_lo = (jnp.dot(e, A_ref[:, :Dh], preferred_element_type=jnp.float32)
            + _lo_bf16(pg_i) + _lo_bf16(pm_i) + b1_ref[:, :Dh])
    h_hi = (jnp.dot(e, A_ref[:, Dh:], preferred_element_type=jnp.float32)
            + _hi_bf16(pg_i) + _hi_bf16(pm_i) + b1_ref[:, Dh:])
    z = (jnp.dot(_silu(h_lo), W2_ref[:Dh, :],
                 preferred_element_type=jnp.float32)
         + jnp.dot(_silu(h_hi), W2_ref[Dh:, :],
                   preferred_element_type=jnp.float32))
    y_ref[...] = _ln(z + b2_ref[...], g_ref[...], bt_ref[...])


def _edge_mlp(e, gi, A, b1, W2, b2, g, bt, lo, bs=3200):
    n = gi.shape[0] // bs
    blk0 = lo // bs
    return pl.pallas_call(
        _edge_body,
        grid=(n,),
        in_specs=[pl.BlockSpec((bs, D), lambda i: (i + blk0, 0)), _rows(bs),
                  _bcast((D, D)), _bcast((1, D)), _bcast((D, D)),
                  _bcast((1, D)), _bcast((1, D)), _bcast((1, D))],
        out_specs=_rows(bs),
        out_shape=jax.ShapeDtypeStruct((gi.shape[0], D), jnp.float32),
        compiler_params=pltpu.CompilerParams(
            dimension_semantics=("arbitrary",)),
    )(e, gi, A, b1.reshape(1, D), W2, b2.reshape(1, D),
      g.reshape(1, D), bt.reshape(1, D))


def _node_body(x_ref, W1_ref, b1_ref, W2_ref, b2_ref, g_ref, bt_ref, o_ref):
    x = x_ref[...]
    h = jnp.dot(x, W1_ref[...], preferred_element_type=jnp.float32)
    h = _silu(h + b1_ref[...])
    z = jnp.dot(h, W2_ref[...], preferred_element_type=jnp.float32)
    o_ref[...] = x + _ln(z + b2_ref[...], g_ref[...], bt_ref[...])


def _grid_mlp(x, W1, b1, W2, b2, g, bt, bs=2000):
    n = x.shape[0] // bs
    return pl.pallas_call(
        _node_body,
        grid=(n,),
        in_specs=[_rows(bs), _bcast((D, D)), _bcast((1, D)), _bcast((D, D)),
                  _bcast((1, D)), _bcast((1, D)), _bcast((1, D))],
        out_specs=_rows(bs),
        out_shape=jax.ShapeDtypeStruct(x.shape, jnp.float32),
        compiler_params=pltpu.CompilerParams(
            dimension_semantics=("arbitrary",)),
    )(x, W1, b1.reshape(1, D), W2, b2.reshape(1, D), g.reshape(1, D),
      bt.reshape(1, D))


def _make_mesh_body(nagg):
    def body(*refs):
        agg_refs = refs[:nagg]
        (x_ref, W1a_ref, W1b_ref, b1_ref, W2_ref, b2_ref, g_ref, bt_ref,
         o_ref) = refs[nagg:]
        a = agg_refs[0][0] + agg_refs[0][1]
        for r in agg_refs[1:]:
            a = a + r[0] + r[1]
        x = x_ref[...]
        h = (jnp.dot(a, W1a_ref[...], preferred_element_type=jnp.float32)
             + jnp.dot(x, W1b_ref[...], preferred_element_type=jnp.float32))
        h = _silu(h + b1_ref[...])
        z = jnp.dot(h, W2_ref[...], preferred_element_type=jnp.float32)
        o_ref[...] = x + _ln(z + b2_ref[...], g_ref[...], bt_ref[...])
    return body


def _mesh_mlp(aggs, x, W1a, W1b, b1, W2, b2, g, bt, bs=2000):
    n = N_MESH // bs
    agg_spec = pl.BlockSpec((2, bs, D), lambda i: (0, i, 0))
    return pl.pallas_call(
        _make_mesh_body(len(aggs)),
        grid=(n,),
        in_specs=[agg_spec] * len(aggs) +
                 [_rows(bs),
                  _bcast((D, D)), _bcast((D, D)), _bcast((1, D)),
                  _bcast((D, D)), _bcast((1, D)), _bcast((1, D)),
                  _bcast((1, D))],
        out_specs=_rows(bs),
        out_shape=jax.ShapeDtypeStruct((N_MESH, D), jnp.float32),
        compiler_params=pltpu.CompilerParams(
            dimension_semantics=("arbitrary",)),
    )(*aggs, x, W1a, W1b, b1.reshape(1, D), W2, b2.reshape(1, D),
      g.reshape(1, D), bt.reshape(1, D))


_CHG = 256


def _sc_gather(pg, pm, ei, lo, size):
    nck = size // _CHG
    Dh = D // 2
    mesh = plsc.VectorSubcoreMesh(core_axis_name="c", subcore_axis_name="s")

    @functools.partial(
        pl.kernel,
        out_type=jax.ShapeDtypeStruct((size, D), jnp.int32),
        mesh=mesh,
        scratch_types=[
            pltpu.VMEM((2, _CHG), jnp.int32),
            pltpu.VMEM((2, _CHG), jnp.int32),
            pltpu.VMEM((_CHG, Dh), jnp.int32),
            pltpu.VMEM((_CHG, Dh), jnp.int32),
            pltpu.SemaphoreType.DMA,
            pltpu.SemaphoreType.DMA,
        ],
        compiler_params=pltpu.CompilerParams(use_tc_tiling_on_sc=False),
    )
    def k(pg_hbm, pm_hbm, ei_hbm, go_hbm, ix0_v, ix1_v, rg_v, rm_v,
          sem_i, sem_g):
        wid = lax.axis_index("s") * _NC + lax.axis_index("c")
        nw = (nck - wid + _NW - 1) // _NW

        pltpu.async_copy(ei_hbm.at[:, pl.ds(lo + wid * _CHG, _CHG)], ix0_v,
                         sem_i)

        def phase(j, cur, nxt):
            @pl.when(j < nw)
            def _():
                ci = wid + j * _NW
                off = ci * _CHG
                pltpu.make_async_copy(
                    ei_hbm.at[:, pl.ds(lo + off, _CHG)], cur, sem_i).wait()

                @pl.when(j + 1 < nw)
                def _():
                    off2 = (ci + _NW) * _CHG
                    pltpu.async_copy(
                        ei_hbm.at[:, pl.ds(lo + off2, _CHG)], nxt, sem_i)

                cs = [
                    pltpu.async_copy(pg_hbm.at[cur.at[0, pl.ds(0, 128)]],
                                     rg_v.at[pl.ds(0, 128)], sem_g),
                    pltpu.async_copy(pg_hbm.at[cur.at[0, pl.ds(128, 128)]],
                                     rg_v.at[pl.ds(128, 128)], sem_g),
                    pltpu.async_copy(pm_hbm.at[cur.at[1, pl.ds(0, 128)]],
                                     rm_v.at[pl.ds(0, 128)], sem_g),
                    pltpu.async_copy(pm_hbm.at[cur.at[1, pl.ds(128, 128)]],
                                     rm_v.at[pl.ds(128, 128)], sem_g),
                ]
                for c in cs:
                    c.wait()
                pltpu.sync_copy(rg_v, go_hbm.at[pl.ds(off, _CHG),
                                                pl.ds(0, Dh)])
                pltpu.sync_copy(rm_v, go_hbm.at[pl.ds(off, _CHG),
                                                pl.ds(Dh, Dh)])

        def pair(i2, carry):
            phase(i2 * 2, ix0_v, ix1_v)
            phase(i2 * 2 + 1, ix1_v, ix0_v)
            return carry

        lax.fori_loop(0, (nw + 1) // 2, pair, 0)

    return k(pg, pm, ei)


def _sc_scatter(ys_los, dst):
    nys = len(ys_los)
    ncks = [y.shape[0] // _CH for y, _ in ys_los]
    los = [lo for _, lo in ys_los]
    mesh = plsc.VectorSubcoreMesh(core_axis_name="c", subcore_axis_name="s")

    @functools.partial(
        pl.kernel,
        out_type=jax.ShapeDtypeStruct((_NC, _NS, _BR, D), jnp.float32),
        mesh=mesh,
        scratch_types=[
            pltpu.VMEM((_CH,), jnp.int32),
            pltpu.VMEM((_CH,), jnp.int32),
            pltpu.VMEM((_CH, D), jnp.float32),
            pltpu.VMEM((_CH, D), jnp.float32),
            pltpu.VMEM((_DR, D), jnp.float32),
            pltpu.VMEM_SHARED((_NPAD, D), jnp.float32),
            pltpu.SemaphoreType.DMA,
            pltpu.SemaphoreType.DMA,
            pltpu.SemaphoreType.DMA,
            pltpu.SemaphoreType.DMA,
        ],
    )
    def k(*refs):
        y_hbms = refs[:nys]
        (dst_hbm, out_hbm, di0_v, di1_v, rw0_v, rw1_v, buf_v,
         acc_sh, semL0, semL1, semA0, semA1) = refs[nys:]
        c = lax.axis_index("c")
        s = lax.axis_index("s")
        wid = s * _NC + c

        zero = jnp.zeros((16,), jnp.float32)

        def zrow(r, carry):
            for j in range(D // 16):
                buf_v[r, pl.ds(j * 16, 16)] = zero
            return carry

        lax.fori_loop(0, _DR, zrow, 0)
        for j in range(_BR // _DR):
            pltpu.sync_copy(buf_v, acc_sh.at[pl.ds(s * _BR + j * _DR, _DR)])
        plsc.subcore_barrier()

        bufs = ((di0_v, rw0_v, semL0, semA0), (di1_v, rw1_v, semL1, semA1))

        def wait_A(b):
            pltpu.make_async_copy(b[1], acc_sh.at[b[0]], b[3]).wait()

        for y_hbm, nck, lo in zip(y_hbms, ncks, los):
            nw = (nck - wid + _NW - 1) // _NW

            def fire_L(j, b):
                off = (wid + j * _NW) * _CH
                pltpu.async_copy(dst_hbm.at[pl.ds(lo + off, _CH)], b[0], b[2])
                pltpu.async_copy(y_hbm.at[pl.ds(off, _CH)], b[1], b[2])

            def wait_L(b):
                pltpu.make_async_copy(
                    dst_hbm.at[pl.ds(lo, _CH)], b[0], b[2]).wait()
                pltpu.make_async_copy(
                    y_hbm.at[pl.ds(0, _CH)], b[1], b[2]).wait()

            fire_L(0, bufs[0])

            def phase(j, p):
                cur = bufs[p]
                oth = bufs[1 - p]

                @pl.when(j < nw)
                def _():
                    wait_L(cur)
                    pltpu.async_copy(cur[1], acc_sh.at[cur[0]], cur[3],
                                     add=True)

                    @pl.when(j + 1 < nw)
                    def _():
                        @pl.when(j >= 1)
                        def _():
                            wait_A(oth)

                        fire_L(j + 1, oth)

            def pair(i2, carry):
                phase(i2 * 2, 0)
                phase(i2 * 2 + 1, 1)
                return carry

            lax.fori_loop(0, (nw + 1) // 2, pair, 0)

            wait_A(bufs[0])
            wait_A(bufs[1])

        plsc.subcore_barrier()

        for j in range(_BR // _DR):
            pltpu.sync_copy(acc_sh.at[pl.ds(s * _BR + j * _DR, _DR)], buf_v)
            pltpu.sync_copy(buf_v, out_hbm.at[c, s, pl.ds(j * _DR, _DR)])

    return k(*[y for y, _ in ys_los], dst).reshape(_NC, _NPAD, D)


def kernel(g2m_efeat, grid_nfeat, mesh_nfeat, edge_index,
           eW1, eb1, eW2, eb2, eg, ebt,
           sW1, sb1, sW2, sb2, sg, sbt,
           dW1, db1, dW2, db2, dg, dbt):
    src = edge_index[0]
    dst = edge_index[1]

    A = eW1[:D]
    B = eW1[D:2 * D]
    C = eW1[2 * D:]

    pgi, pmi = _project(grid_nfeat[:N_MESH], mesh_nfeat, B, C)
    splits = (76800, 76800, 76800, 89600)
    lo = 0
    ys = []
    for sz in splits:
        gi_k = _sc_gather(pgi, pmi, edge_index, lo, sz)
        y_k = _edge_mlp(g2m_efeat, gi_k, A, eb1, eW2, eb2, eg, ebt, lo)
        ys.append((y_k, lo))
        lo += sz
    aggs = [_sc_scatter([p], dst) for p in ys]
    mesh_new = _mesh_mlp(aggs, mesh_nfeat, dW1[:D], dW1[D:], db1,
                         dW2, db2, dg, dbt)
    grid_new = _grid_mlp(grid_nfeat, sW1, sb1, sW2, sb2, sg, sbt)
    return (grid_new, mesh_new)
